# baseline (speedup 1.0000x reference)
from functools import partial

import numpy as np

import jax
import jax.numpy as jnp
from jax.experimental import pallas as pl
from jax.experimental.pallas import tpu as pltpu


def _ca_flat_kernel(x_ref, g1_ref, g2_ref, ts_ref, w1t_ref, b1_ref, w2t_ref,
                    b2_ref, o_ref, *, inv_hw):
    X = x_ref[0]
    R = X.shape[0]

    li = jax.lax.broadcasted_iota(jnp.int32, (128, 2), 0)
    ci = jax.lax.broadcasted_iota(jnp.int32, (128, 2), 1)
    halves = jnp.where((li < 64) == (ci == 0), 1.0, 0.0)
    rlu = jax.lax.dot_general(X, halves, (((1,), (0,)), ((), ())),
                              preferred_element_type=jnp.float32)

    s1 = jax.lax.dot_general(rlu[:, 0:1], g1_ref[...],
                             (((0,), (0,)), ((), ())),
                             preferred_element_type=jnp.float32)
    s2 = jax.lax.dot_general(rlu[:, 1:2], g2_ref[...],
                             (((0,), (0,)), ((), ())),
                             preferred_element_type=jnp.float32)
    y = (s1 + s2) * inv_hw

    t1 = jnp.dot(y, w1t_ref[...], preferred_element_type=jnp.float32)
    t1 = jnp.maximum(t1 + b1_ref[...], 0.0)
    t2 = jnp.dot(t1, w2t_ref[...], preferred_element_type=jnp.float32)
    scale = jax.nn.sigmoid(t2 + b2_ref[...])

    a = jax.lax.dot_general(g1_ref[...], scale, (((1,), (1,)), ((), ())),
                            preferred_element_type=jnp.float32)
    b = jax.lax.dot_general(g2_ref[...], scale, (((1,), (1,)), ((), ())),
                            preferred_element_type=jnp.float32)
    lane = jax.lax.broadcasted_iota(jnp.int32, (R, 128), 1)
    sfull = jnp.where(lane < ts_ref[...], a, b)
    o_ref[0] = x_ref[0] * sfull


def _flat_consts(C, HW):
    R = C * HW // 128
    r = np.arange(R)
    cl = (128 * r) // HW
    cu = (128 * r + 127) // HW
    g1 = (cl[:, None] == np.arange(C)[None, :]).astype(np.float32)
    g2 = (cu[:, None] == np.arange(C)[None, :]).astype(np.float32)
    tsplit = np.minimum(HW * (cl + 1) - 128 * r, 128).astype(np.int32)
    return g1, g2, tsplit.reshape(R, 1)


def _ca_chan_kernel(x_ref, w1t_ref, b1_ref, w2t_ref, b2_ref, o_ref, *,
                    inv_hw, hw):
    x = x_ref[...]
    if hw % 128 != 0:
        lane = jax.lax.broadcasted_iota(jnp.int32, x.shape, dimension=2)
        x = jnp.where(lane < hw, x, 0.0)
    y = jnp.sum(x, axis=-1, dtype=jnp.float32) * inv_hw
    t1 = jnp.dot(y, w1t_ref[...], preferred_element_type=jnp.float32)
    t1 = jnp.maximum(t1 + b1_ref[...], 0.0)
    t2 = jnp.dot(t1, w2t_ref[...], preferred_element_type=jnp.float32)
    scale = jax.nn.sigmoid(t2 + b2_ref[...]).astype(x_ref.dtype)
    o_ref[...] = (x_ref[...] * scale[:, :, None]).astype(o_ref.dtype)


def kernel(x, w1, b1, w2, b2):
    B, C, H, W = x.shape
    HW = H * W
    Cr = w1.shape[0]
    inv_hw = float(1.0 / HW)

    w1t = jnp.transpose(w1)
    w2t = jnp.transpose(w2)
    b1r = b1.reshape(1, Cr)
    b2r = b2.reshape(1, C)

    total = C * HW
    if total % 128 == 0 and HW % 64 == 0 and HW >= 128:
        R = total // 128
        g1, g2, tsplit = _flat_consts(C, HW)
        x_flat = x.reshape(B, R, 128)
        out_flat = pl.pallas_call(
            partial(_ca_flat_kernel, inv_hw=inv_hw),
            out_shape=jax.ShapeDtypeStruct((B, R, 128), x.dtype),
            grid=(B,),
            in_specs=[
                pl.BlockSpec((1, R, 128), lambda b: (b, 0, 0)),
                pl.BlockSpec((R, C), lambda b: (0, 0)),
                pl.BlockSpec((R, C), lambda b: (0, 0)),
                pl.BlockSpec((R, 1), lambda b: (0, 0)),
                pl.BlockSpec((C, Cr), lambda b: (0, 0)),
                pl.BlockSpec((1, Cr), lambda b: (0, 0)),
                pl.BlockSpec((Cr, C), lambda b: (0, 0)),
                pl.BlockSpec((1, C), lambda b: (0, 0)),
            ],
            out_specs=pl.BlockSpec((1, R, 128), lambda b: (b, 0, 0)),
            compiler_params=pltpu.CompilerParams(
                dimension_semantics=("parallel",),
                vmem_limit_bytes=56 * 1024 * 1024,
            ),
        )(x_flat, jnp.asarray(g1), jnp.asarray(g2), jnp.asarray(tsplit),
          w1t, b1r, w2t, b2r)
        return out_flat.reshape(B, C, H, W)

    x_flat = x.reshape(B, C, HW)
    out_flat = pl.pallas_call(
        partial(_ca_chan_kernel, inv_hw=inv_hw, hw=HW),
        out_shape=jax.ShapeDtypeStruct((B, C, HW), x.dtype),
        grid=(B,),
        in_specs=[
            pl.BlockSpec((1, C, HW), lambda b: (b, 0, 0)),
            pl.BlockSpec((C, Cr), lambda b: (0, 0)),
            pl.BlockSpec((1, Cr), lambda b: (0, 0)),
            pl.BlockSpec((Cr, C), lambda b: (0, 0)),
            pl.BlockSpec((1, C), lambda b: (0, 0)),
        ],
        out_specs=pl.BlockSpec((1, C, HW), lambda b: (b, 0, 0)),
        compiler_params=pltpu.CompilerParams(
            dimension_semantics=("parallel",),
            vmem_limit_bytes=48 * 1024 * 1024,
        ),
    )(x_flat, w1t, b1r, w2t, b2r)
    return out_flat.reshape(B, C, H, W)


# speedup vs baseline: 11.4281x; 11.4281x over previous
from functools import partial

import jax
import jax.numpy as jnp
from jax.experimental import pallas as pl
from jax.experimental.pallas import tpu as pltpu


def _ca_nhwc_kernel(x_ref, w1t_ref, b1_ref, w2t_ref, b2_ref, o_ref, *, inv_hw):
    X = x_ref[0]
    y = jnp.sum(X, axis=0, keepdims=True, dtype=jnp.float32) * inv_hw

    t1 = jnp.dot(y, w1t_ref[...], preferred_element_type=jnp.float32)
    t1 = jnp.maximum(t1 + b1_ref[...], 0.0)
    t2 = jnp.dot(t1, w2t_ref[...], preferred_element_type=jnp.float32)
    scale = jax.nn.sigmoid(t2 + b2_ref[...]).astype(x_ref.dtype)

    o_ref[0] = (x_ref[0] * scale).astype(o_ref.dtype)


def _ca_chan_kernel(x_ref, w1t_ref, b1_ref, w2t_ref, b2_ref, o_ref, *,
                    inv_hw, hw):
    x = x_ref[...]
    if hw % 128 != 0:
        lane = jax.lax.broadcasted_iota(jnp.int32, x.shape, dimension=2)
        x = jnp.where(lane < hw, x, 0.0)
    y = jnp.sum(x, axis=-1, dtype=jnp.float32) * inv_hw
    t1 = jnp.dot(y, w1t_ref[...], preferred_element_type=jnp.float32)
    t1 = jnp.maximum(t1 + b1_ref[...], 0.0)
    t2 = jnp.dot(t1, w2t_ref[...], preferred_element_type=jnp.float32)
    scale = jax.nn.sigmoid(t2 + b2_ref[...]).astype(x_ref.dtype)
    o_ref[...] = (x_ref[...] * scale[:, :, None]).astype(o_ref.dtype)


def kernel(x, w1, b1, w2, b2):
    B, C, H, W = x.shape
    HW = H * W
    Cr = w1.shape[0]
    inv_hw = float(1.0 / HW)

    w1t = jnp.transpose(w1)
    w2t = jnp.transpose(w2)
    b1r = b1.reshape(1, Cr)
    b2r = b2.reshape(1, C)

    if W % 8 == 0 and C % 128 == 0:
        x_nhwc = jnp.transpose(x, (0, 2, 3, 1)).reshape(B, HW, C)
        out_nhwc = pl.pallas_call(
            partial(_ca_nhwc_kernel, inv_hw=inv_hw),
            out_shape=jax.ShapeDtypeStruct((B, HW, C), x.dtype),
            grid=(B,),
            in_specs=[
                pl.BlockSpec((1, HW, C), lambda b: (b, 0, 0)),
                pl.BlockSpec((C, Cr), lambda b: (0, 0)),
                pl.BlockSpec((1, Cr), lambda b: (0, 0)),
                pl.BlockSpec((Cr, C), lambda b: (0, 0)),
                pl.BlockSpec((1, C), lambda b: (0, 0)),
            ],
            out_specs=pl.BlockSpec((1, HW, C), lambda b: (b, 0, 0)),
            compiler_params=pltpu.CompilerParams(
                dimension_semantics=("parallel",),
                vmem_limit_bytes=48 * 1024 * 1024,
            ),
        )(x_nhwc, w1t, b1r, w2t, b2r)
        return jnp.transpose(out_nhwc.reshape(B, H, W, C), (0, 3, 1, 2))

    x_flat = x.reshape(B, C, HW)
    out_flat = pl.pallas_call(
        partial(_ca_chan_kernel, inv_hw=inv_hw, hw=HW),
        out_shape=jax.ShapeDtypeStruct((B, C, HW), x.dtype),
        grid=(B,),
        in_specs=[
            pl.BlockSpec((1, C, HW), lambda b: (b, 0, 0)),
            pl.BlockSpec((C, Cr), lambda b: (0, 0)),
            pl.BlockSpec((1, Cr), lambda b: (0, 0)),
            pl.BlockSpec((Cr, C), lambda b: (0, 0)),
            pl.BlockSpec((1, C), lambda b: (0, 0)),
        ],
        out_specs=pl.BlockSpec((1, C, HW), lambda b: (b, 0, 0)),
        compiler_params=pltpu.CompilerParams(
            dimension_semantics=("parallel",),
            vmem_limit_bytes=48 * 1024 * 1024,
        ),
    )(x_flat, w1t, b1r, w2t, b2r)
    return out_flat.reshape(B, C, H, W)


# speedup vs baseline: 12.3104x; 1.0772x over previous
from functools import partial

import jax
import jax.numpy as jnp
from jax.experimental import pallas as pl
from jax.experimental.pallas import tpu as pltpu


def _ca_nhwc_kernel(x_ref, w1t_ref, b1_ref, w2t_ref, b2_ref, o_ref, *, inv_hw):
    y = jnp.sum(x_ref[...], axis=1, dtype=jnp.float32) * inv_hw

    t1 = jnp.dot(y, w1t_ref[...], preferred_element_type=jnp.float32)
    t1 = jnp.maximum(t1 + b1_ref[...], 0.0)
    t2 = jnp.dot(t1, w2t_ref[...], preferred_element_type=jnp.float32)
    scale = jax.nn.sigmoid(t2 + b2_ref[...]).astype(x_ref.dtype)

    o_ref[...] = (x_ref[...] * scale[:, None, :]).astype(o_ref.dtype)


def _ca_chan_kernel(x_ref, w1t_ref, b1_ref, w2t_ref, b2_ref, o_ref, *,
                    inv_hw, hw):
    x = x_ref[...]
    if hw % 128 != 0:
        lane = jax.lax.broadcasted_iota(jnp.int32, x.shape, dimension=2)
        x = jnp.where(lane < hw, x, 0.0)
    y = jnp.sum(x, axis=-1, dtype=jnp.float32) * inv_hw
    t1 = jnp.dot(y, w1t_ref[...], preferred_element_type=jnp.float32)
    t1 = jnp.maximum(t1 + b1_ref[...], 0.0)
    t2 = jnp.dot(t1, w2t_ref[...], preferred_element_type=jnp.float32)
    scale = jax.nn.sigmoid(t2 + b2_ref[...]).astype(x_ref.dtype)
    o_ref[...] = (x_ref[...] * scale[:, :, None]).astype(o_ref.dtype)


def kernel(x, w1, b1, w2, b2):
    B, C, H, W = x.shape
    HW = H * W
    Cr = w1.shape[0]
    inv_hw = float(1.0 / HW)

    w1t = jnp.transpose(w1)
    w2t = jnp.transpose(w2)
    b1r = b1.reshape(1, Cr)
    b2r = b2.reshape(1, C)

    if W % 8 == 0 and C % 128 == 0:
        x_nhwc = jnp.transpose(x, (0, 2, 3, 1)).reshape(B, HW, C)
        out_nhwc = pl.pallas_call(
            partial(_ca_nhwc_kernel, inv_hw=inv_hw),
            out_shape=jax.ShapeDtypeStruct((B, HW, C), x.dtype),
            grid=(B // 2,) if B % 2 == 0 else (B,),
            in_specs=[
                pl.BlockSpec((2 if B % 2 == 0 else 1, HW, C),
                             lambda b: (b, 0, 0)),
                pl.BlockSpec((C, Cr), lambda b: (0, 0)),
                pl.BlockSpec((1, Cr), lambda b: (0, 0)),
                pl.BlockSpec((Cr, C), lambda b: (0, 0)),
                pl.BlockSpec((1, C), lambda b: (0, 0)),
            ],
            out_specs=pl.BlockSpec((2 if B % 2 == 0 else 1, HW, C),
                                   lambda b: (b, 0, 0)),
            compiler_params=pltpu.CompilerParams(
                dimension_semantics=("parallel",),
                vmem_limit_bytes=48 * 1024 * 1024,
            ),
        )(x_nhwc, w1t, b1r, w2t, b2r)
        return jnp.transpose(out_nhwc.reshape(B, H, W, C), (0, 3, 1, 2))

    x_flat = x.reshape(B, C, HW)
    out_flat = pl.pallas_call(
        partial(_ca_chan_kernel, inv_hw=inv_hw, hw=HW),
        out_shape=jax.ShapeDtypeStruct((B, C, HW), x.dtype),
        grid=(B,),
        in_specs=[
            pl.BlockSpec((1, C, HW), lambda b: (b, 0, 0)),
            pl.BlockSpec((C, Cr), lambda b: (0, 0)),
            pl.BlockSpec((1, Cr), lambda b: (0, 0)),
            pl.BlockSpec((Cr, C), lambda b: (0, 0)),
            pl.BlockSpec((1, C), lambda b: (0, 0)),
        ],
        out_specs=pl.BlockSpec((1, C, HW), lambda b: (b, 0, 0)),
        compiler_params=pltpu.CompilerParams(
            dimension_semantics=("parallel",),
            vmem_limit_bytes=48 * 1024 * 1024,
        ),
    )(x_flat, w1t, b1r, w2t, b2r)
    return out_flat.reshape(B, C, H, W)


# speedup vs baseline: 12.9830x; 1.0546x over previous
from functools import partial

import jax
import jax.numpy as jnp
from jax.experimental import pallas as pl
from jax.experimental.pallas import tpu as pltpu


def _ca_nhwc_kernel(x_ref, w1t_ref, b1_ref, w2t_ref, b2_ref, o_ref, *, inv_hw):
    y = jnp.sum(x_ref[...], axis=1, dtype=jnp.float32) * inv_hw

    t1 = jnp.dot(y, w1t_ref[...], preferred_element_type=jnp.float32)
    t1 = jnp.maximum(t1 + b1_ref[...], 0.0)
    t2 = jnp.dot(t1, w2t_ref[...], preferred_element_type=jnp.float32)
    scale = jax.nn.sigmoid(t2 + b2_ref[...]).astype(x_ref.dtype)

    o_ref[...] = (x_ref[...] * scale[:, None, :]).astype(o_ref.dtype)


def _ca_chan_kernel(x_ref, w1t_ref, b1_ref, w2t_ref, b2_ref, o_ref, *,
                    inv_hw, hw):
    x = x_ref[...]
    if hw % 128 != 0:
        lane = jax.lax.broadcasted_iota(jnp.int32, x.shape, dimension=2)
        x = jnp.where(lane < hw, x, 0.0)
    y = jnp.sum(x, axis=-1, dtype=jnp.float32) * inv_hw
    t1 = jnp.dot(y, w1t_ref[...], preferred_element_type=jnp.float32)
    t1 = jnp.maximum(t1 + b1_ref[...], 0.0)
    t2 = jnp.dot(t1, w2t_ref[...], preferred_element_type=jnp.float32)
    scale = jax.nn.sigmoid(t2 + b2_ref[...]).astype(x_ref.dtype)
    o_ref[...] = (x_ref[...] * scale[:, :, None]).astype(o_ref.dtype)


def kernel(x, w1, b1, w2, b2):
    B, C, H, W = x.shape
    HW = H * W
    Cr = w1.shape[0]
    inv_hw = float(1.0 / HW)

    w1t = jnp.transpose(w1)
    w2t = jnp.transpose(w2)
    b1r = b1.reshape(1, Cr)
    b2r = b2.reshape(1, C)

    if W % 8 == 0 and C % 128 == 0:
        x_nhwc = jnp.transpose(x, (0, 2, 3, 1)).reshape(B, HW, C)
        out_nhwc = pl.pallas_call(
            partial(_ca_nhwc_kernel, inv_hw=inv_hw),
            out_shape=jax.ShapeDtypeStruct((B, HW, C), x.dtype),
            grid=(B // 4,) if B % 4 == 0 else (B,),
            in_specs=[
                pl.BlockSpec((4 if B % 4 == 0 else 1, HW, C),
                             lambda b: (b, 0, 0)),
                pl.BlockSpec((C, Cr), lambda b: (0, 0)),
                pl.BlockSpec((1, Cr), lambda b: (0, 0)),
                pl.BlockSpec((Cr, C), lambda b: (0, 0)),
                pl.BlockSpec((1, C), lambda b: (0, 0)),
            ],
            out_specs=pl.BlockSpec((4 if B % 4 == 0 else 1, HW, C),
                                   lambda b: (b, 0, 0)),
            compiler_params=pltpu.CompilerParams(
                dimension_semantics=("parallel",),
                vmem_limit_bytes=56 * 1024 * 1024,
            ),
        )(x_nhwc, w1t, b1r, w2t, b2r)
        return jnp.transpose(out_nhwc.reshape(B, H, W, C), (0, 3, 1, 2))

    x_flat = x.reshape(B, C, HW)
    out_flat = pl.pallas_call(
        partial(_ca_chan_kernel, inv_hw=inv_hw, hw=HW),
        out_shape=jax.ShapeDtypeStruct((B, C, HW), x.dtype),
        grid=(B,),
        in_specs=[
            pl.BlockSpec((1, C, HW), lambda b: (b, 0, 0)),
            pl.BlockSpec((C, Cr), lambda b: (0, 0)),
            pl.BlockSpec((1, Cr), lambda b: (0, 0)),
            pl.BlockSpec((Cr, C), lambda b: (0, 0)),
            pl.BlockSpec((1, C), lambda b: (0, 0)),
        ],
        out_specs=pl.BlockSpec((1, C, HW), lambda b: (b, 0, 0)),
        compiler_params=pltpu.CompilerParams(
            dimension_semantics=("parallel",),
            vmem_limit_bytes=48 * 1024 * 1024,
        ),
    )(x_flat, w1t, b1r, w2t, b2r)
    return out_flat.reshape(B, C, H, W)
